# baseline (speedup 1.0000x reference)
import functools

import jax
import jax.numpy as jnp
from jax import lax
from jax.experimental import pallas as pl
from jax.experimental.pallas import tpu as pltpu
from jax.experimental.pallas import tpu_sc as plsc

N = 10000
E = 320000
D = 128
G = 16
C = 10

NP = 10240
NC = 2
NS = 16
NW = NC * NS
EPW = E // NW
CHUNK = 128
NFULL = EPW // CHUNK
TAIL = EPW - NFULL * CHUNK
DEPTH = 2
ROWS_PER_TILE = NP // NS

RBLK = 1024
NB = NP // RBLK
PAD_GRAPH = 127


def _sc_aggregate(x, epack, srcr, dstr, zeros_pad, zeros_flat):
    mesh = plsc.VectorSubcoreMesh(core_axis_name="c", subcore_axis_name="s")

    @functools.partial(
        pl.kernel,
        out_type=(jax.ShapeDtypeStruct((NC, NP, D), jnp.float32),
                  jax.ShapeDtypeStruct((NW, NP), jnp.float32)),
        mesh=mesh,
        scratch_types=[
            pltpu.VMEM_SHARED((NP, D), jnp.float32),
            [pltpu.VMEM((2 * CHUNK,), jnp.int32) for _ in range(2 * DEPTH)],
            [pltpu.VMEM((CHUNK, D), jnp.float32) for _ in range(DEPTH)],
            pltpu.VMEM((TAIL,), jnp.int32),
            pltpu.VMEM((TAIL,), jnp.int32),
            pltpu.VMEM((TAIL, D), jnp.float32),
            pltpu.VMEM((NP,), jnp.float32),
            [pltpu.SemaphoreType.DMA for _ in range(DEPTH)],
            [pltpu.SemaphoreType.DMA for _ in range(DEPTH)],
            [pltpu.SemaphoreType.DMA for _ in range(DEPTH)],
            pltpu.SemaphoreType.DMA,
        ],
        compiler_params=pltpu.CompilerParams(needs_layout_passes=False),
    )
    def k(x_hbm, ep_hbm, src_hbm, dst_hbm, zeros_hbm, zflat_hbm,
          agg_out, deg_out,
          a_sh, ibuf, rows, srct, dstt, rowst, degloc,
          semg, sems, sema, semt):
        c = lax.axis_index("c")
        s = lax.axis_index("s")
        wid = s * NC + c
        ebase = wid * EPW
        cbase = wid * NFULL * 2 * CHUNK
        r0 = s * ROWS_PER_TILE
        pltpu.sync_copy(zeros_hbm, a_sh.at[pl.ds(r0, ROWS_PER_TILE)])
        pltpu.sync_copy(zflat_hbm, degloc)
        for b in range(DEPTH):
            pltpu.sync_copy(
                ep_hbm.at[pl.ds(cbase + b * 2 * CHUNK, 2 * CHUNK)],
                ibuf[b])
            pltpu.async_copy(x_hbm.at[ibuf[b].at[pl.ds(CHUNK, CHUNK)]],
                             rows[b], semg[b])
        plsc.subcore_barrier()

        ones16 = jnp.ones((16,), jnp.float32)

        NI = 2 * DEPTH

        def hist(ib):
            for j in range(CHUNK // 16):
                d16 = ibuf[ib][pl.ds(j * 16, 16)]
                plsc.addupdate_scatter(degloc, [d16], ones16)

        def stage(ck, j, prefetch):
            b = j % DEPTH
            ib = j % NI
            nib = (j + DEPTH) % NI
            src_idx = ibuf[ib].at[pl.ds(CHUNK, CHUNK)]
            dst_idx = ibuf[ib].at[pl.ds(0, CHUNK)]
            pltpu.make_async_copy(x_hbm.at[src_idx], rows[b],
                                  semg[b]).wait()
            sdesc = pltpu.async_copy(rows[b], a_sh.at[dst_idx],
                                     sems[b], add=True)
            if prefetch:
                adesc = pltpu.async_copy(
                    ep_hbm.at[pl.ds(cbase + (ck + DEPTH) * 2 * CHUNK,
                                    2 * CHUNK)], ibuf[nib], sema[b])
            hist(ib)
            sdesc.wait()
            if prefetch:
                adesc.wait()
                pltpu.async_copy(x_hbm.at[ibuf[nib].at[pl.ds(CHUNK, CHUNK)]],
                                 rows[b], semg[b])

        def body(ii, carry):
            for j in range(NI):
                stage(NI * ii + j, j, True)
            return carry

        nmain = NFULL // NI - 1
        lax.fori_loop(0, nmain, body, 0)
        for ck in range(nmain * NI, NFULL):
            stage(ck, ck % NI, ck + DEPTH < NFULL)
        tbase = ebase + NFULL * CHUNK
        pltpu.sync_copy(src_hbm.at[pl.ds(tbase, TAIL)], srct)
        pltpu.sync_copy(dst_hbm.at[pl.ds(tbase, TAIL)], dstt)
        pltpu.async_copy(x_hbm.at[srct], rowst, semt).wait()
        pltpu.sync_copy(rowst, a_sh.at[dstt], add=True)
        plsc.addupdate_scatter(degloc, [dstt[...]], ones16)
        plsc.subcore_barrier()
        pltpu.sync_copy(a_sh.at[pl.ds(r0, ROWS_PER_TILE)],
                        agg_out.at[c, pl.ds(r0, ROWS_PER_TILE)])
        pltpu.sync_copy(degloc, deg_out.at[wid])

    return k(x, epack, srcr, dstr, zeros_pad, zeros_flat)


def _tc_body(x_ref, p_ref, d_ref, b_ref, wn_ref, w1_ref, w2_ref,
             b1_ref, b2_ref, o_ref, accp, accc):
    i = pl.program_id(0)

    @pl.when(i == 0)
    def _init():
        accp[...] = jnp.zeros_like(accp)
        accc[...] = jnp.zeros_like(accc)

    agg = p_ref[0] + p_ref[1]
    dsum = jnp.sum(d_ref[...], axis=0)
    r = 1.0 / jnp.maximum(dsum, 1.0)
    rmat = r.reshape(RBLK // 128, 128)
    eye = (lax.broadcasted_iota(jnp.int32, (128, 128), 0)
           == lax.broadcasted_iota(jnp.int32, (128, 128), 1))
    parts = []
    for t in range(RBLK // 128):
        dt = jnp.where(eye, rmat[t:t + 1, :], 0.0)
        parts.append(jnp.dot(dt, agg[128 * t:128 * (t + 1), :],
                             preferred_element_type=jnp.float32))
    aggn = jnp.concatenate(parts, axis=0)
    h = (x_ref[...]
         + jnp.dot(aggn, wn_ref[...], preferred_element_type=jnp.float32))
    h = jnp.maximum(h, 0.0)

    b = b_ref[0]
    gr = lax.broadcasted_iota(jnp.int32, (128, RBLK), 0)
    oh = (gr == b).astype(jnp.float32)
    accp[...] += jnp.dot(oh, h, preferred_element_type=jnp.float32)
    accc[...] += jnp.dot(oh, jnp.ones((RBLK, 128), jnp.float32),
                         preferred_element_type=jnp.float32)

    @pl.when(i == NB - 1)
    def _final():
        pooled = accp[...] / jnp.maximum(accc[...], 1.0)
        z1 = jnp.maximum(
            jnp.dot(pooled, w1_ref[...], preferred_element_type=jnp.float32)
            + b1_ref[...], 0.0)
        z = (jnp.dot(z1, w2_ref[...], preferred_element_type=jnp.float32)
             + b2_ref[...])
        lanes = lax.broadcasted_iota(jnp.int32, (128, 128), 1)
        z = jnp.where(lanes < C, z, -1e30)
        m = jnp.max(z, axis=1, keepdims=True)
        e = jnp.exp(z - m)
        sm = e / jnp.sum(e, axis=1, keepdims=True)
        o_ref[...] = sm[0:G, :]


def _xw_body(x_ref, w_ref, b_ref, o_ref):
    o_ref[...] = (jnp.dot(x_ref[...], w_ref[...],
                          preferred_element_type=jnp.float32) + b_ref[...])


def _tc_root(x_pad, W_root, b_gnn):
    return pl.pallas_call(
        _xw_body,
        grid=(NB,),
        in_specs=[
            pl.BlockSpec((RBLK, D), lambda i: (i, 0)),
            pl.BlockSpec((D, D), lambda i: (0, 0)),
            pl.BlockSpec((1, D), lambda i: (0, 0)),
        ],
        out_specs=pl.BlockSpec((RBLK, D), lambda i: (i, 0)),
        out_shape=jax.ShapeDtypeStruct((NP, D), jnp.float32),
    )(x_pad, W_root, b_gnn)


def _tc_classify(xr, aggp, degp, batch3, W_nbr, W1, b1, W2p, b2p):
    return pl.pallas_call(
        _tc_body,
        grid=(NB,),
        in_specs=[
            pl.BlockSpec((RBLK, D), lambda i: (i, 0)),
            pl.BlockSpec((NC, RBLK, D), lambda i: (0, i, 0)),
            pl.BlockSpec((NW, RBLK), lambda i: (0, i)),
            pl.BlockSpec((1, 1, RBLK), lambda i: (i, 0, 0)),
            pl.BlockSpec((D, D), lambda i: (0, 0)),
            pl.BlockSpec((D, D), lambda i: (0, 0)),
            pl.BlockSpec((D, 128), lambda i: (0, 0)),
            pl.BlockSpec((1, D), lambda i: (0, 0)),
            pl.BlockSpec((1, 128), lambda i: (0, 0)),
        ],
        out_specs=pl.BlockSpec((G, 128), lambda i: (0, 0)),
        out_shape=jax.ShapeDtypeStruct((G, 128), jnp.float32),
        scratch_shapes=[
            pltpu.VMEM((128, 128), jnp.float32),
            pltpu.VMEM((128, 128), jnp.float32),
        ],
    )(xr, aggp, degp, batch3, W_nbr, W1, W2p, b1, b2p)


def kernel(x, edge_index, batch, W_root, W_nbr, b_gnn, W1, b1, W2, b2):
    zeros_pad = jnp.zeros((ROWS_PER_TILE, D), jnp.float32)
    zeros_flat = jnp.zeros((NP,), jnp.float32)
    srcf = edge_index[0].reshape(NW, EPW)[:, :NFULL * CHUNK]
    dstf = edge_index[1].reshape(NW, EPW)[:, :NFULL * CHUNK]
    epack = jnp.stack([dstf.reshape(NW, NFULL, CHUNK),
                       srcf.reshape(NW, NFULL, CHUNK)],
                      axis=2).reshape(NW * NFULL * 2 * CHUNK)

    aggp, degp = _sc_aggregate(x, epack, edge_index[0], edge_index[1],
                               zeros_pad, zeros_flat)

    x_pad = jnp.pad(x, ((0, NP - N), (0, 0)))
    xr = _tc_root(x_pad, W_root, b_gnn.reshape(1, D))
    batch3 = jnp.pad(batch, (0, NP - N),
                     constant_values=PAD_GRAPH).reshape(NB, 1, RBLK)
    W2p = jnp.pad(W2, ((0, 0), (0, 128 - C)))
    b2p = jnp.pad(b2, (0, 128 - C)).reshape(1, 128)
    out = _tc_classify(xr, aggp, degp, batch3, W_nbr,
                       W1, b1.reshape(1, D), W2p, b2p)
    return out[:, :C]


# speedup vs baseline: 1.0226x; 1.0226x over previous
import functools

import jax
import jax.numpy as jnp
from jax import lax
from jax.experimental import pallas as pl
from jax.experimental.pallas import tpu as pltpu
from jax.experimental.pallas import tpu_sc as plsc

N = 10000
E = 320000
D = 128
G = 16
C = 10

NP = 10240
NC = 2
NS = 16
NW = NC * NS
EPW = E // NW
CHUNK = 128
NFULL = EPW // CHUNK
TAIL = EPW - NFULL * CHUNK
DEPTH = 2
ROWS_PER_TILE = NP // NS

RBLK = 1024
NB = NP // RBLK
PAD_GRAPH = 127


def _sc_aggregate(x, srcr, dstr, zeros_pad, zeros_flat):
    mesh = plsc.VectorSubcoreMesh(core_axis_name="c", subcore_axis_name="s")

    @functools.partial(
        pl.kernel,
        out_type=(jax.ShapeDtypeStruct((NC, NP, D), jnp.float32),
                  jax.ShapeDtypeStruct((NW, NP), jnp.float32)),
        mesh=mesh,
        scratch_types=[
            pltpu.VMEM_SHARED((NP, D), jnp.float32),
            [pltpu.VMEM((CHUNK,), jnp.int32) for _ in range(DEPTH)],
            [pltpu.VMEM((CHUNK,), jnp.int32) for _ in range(DEPTH)],
            [pltpu.VMEM((CHUNK, D), jnp.float32) for _ in range(DEPTH)],
            pltpu.VMEM((TAIL,), jnp.int32),
            pltpu.VMEM((TAIL,), jnp.int32),
            pltpu.VMEM((TAIL, D), jnp.float32),
            pltpu.VMEM((NP,), jnp.float32),
            [pltpu.SemaphoreType.DMA for _ in range(DEPTH)],
            [pltpu.SemaphoreType.DMA for _ in range(DEPTH)],
            [pltpu.SemaphoreType.DMA for _ in range(DEPTH)],
            [pltpu.SemaphoreType.DMA for _ in range(DEPTH)],
            pltpu.SemaphoreType.DMA,
        ],
        compiler_params=pltpu.CompilerParams(needs_layout_passes=False),
    )
    def k(x_hbm, src_hbm, dst_hbm, zeros_hbm, zflat_hbm, agg_out, deg_out,
          a_sh, srcb, dstb, rows, srct, dstt, rowst, degloc,
          semg, sems, sema, semd, semt):
        c = lax.axis_index("c")
        s = lax.axis_index("s")
        wid = s * NC + c
        ebase = wid * EPW
        r0 = s * ROWS_PER_TILE
        pltpu.sync_copy(zeros_hbm, a_sh.at[pl.ds(r0, ROWS_PER_TILE)])
        pltpu.sync_copy(zflat_hbm, degloc)
        for b in range(DEPTH):
            pltpu.sync_copy(src_hbm.at[pl.ds(ebase + b * CHUNK, CHUNK)],
                            srcb[b])
            pltpu.sync_copy(dst_hbm.at[pl.ds(ebase + b * CHUNK, CHUNK)],
                            dstb[b])
            pltpu.async_copy(x_hbm.at[srcb[b]], rows[b], semg[b])
        plsc.subcore_barrier()

        ones16 = jnp.ones((16,), jnp.float32)

        def hist(b):
            for j in range(CHUNK // 16):
                d16 = dstb[b][pl.ds(j * 16, 16)]
                plsc.addupdate_scatter(degloc, [d16], ones16)

        def stage(ck, b, prefetch):
            pltpu.make_async_copy(x_hbm.at[srcb[b]], rows[b],
                                  semg[b]).wait()
            sdesc = pltpu.async_copy(rows[b], a_sh.at[dstb[b]],
                                     sems[b], add=True)
            if prefetch:
                adesc = pltpu.async_copy(
                    src_hbm.at[pl.ds(ebase + (ck + DEPTH) * CHUNK, CHUNK)],
                    srcb[b], sema[b])
            hist(b)
            sdesc.wait()
            if prefetch:
                ddesc = pltpu.async_copy(
                    dst_hbm.at[pl.ds(ebase + (ck + DEPTH) * CHUNK, CHUNK)],
                    dstb[b], semd[b])
                adesc.wait()
                pltpu.async_copy(x_hbm.at[srcb[b]], rows[b], semg[b])
                ddesc.wait()

        def body(ii, carry):
            for b in range(DEPTH):
                stage(DEPTH * ii + b, b, True)
            return carry

        nmain = NFULL // DEPTH - 1
        lax.fori_loop(0, nmain, body, 0)
        for ck in range(nmain * DEPTH, NFULL):
            stage(ck, ck % DEPTH, ck + DEPTH < NFULL)
        tbase = ebase + NFULL * CHUNK
        pltpu.sync_copy(src_hbm.at[pl.ds(tbase, TAIL)], srct)
        pltpu.sync_copy(dst_hbm.at[pl.ds(tbase, TAIL)], dstt)
        pltpu.async_copy(x_hbm.at[srct], rowst, semt).wait()
        pltpu.sync_copy(rowst, a_sh.at[dstt], add=True)
        plsc.addupdate_scatter(degloc, [dstt[...]], ones16)
        plsc.subcore_barrier()
        pltpu.sync_copy(a_sh.at[pl.ds(r0, ROWS_PER_TILE)],
                        agg_out.at[c, pl.ds(r0, ROWS_PER_TILE)])
        pltpu.sync_copy(degloc, deg_out.at[wid])

    return k(x, srcr, dstr, zeros_pad, zeros_flat)


def _tc_body(x_ref, p_ref, d_ref, b_ref, wr_ref, wn_ref, w1_ref, w2_ref,
             bg_ref, b1_ref, b2_ref, o_ref, accp, accc):
    i = pl.program_id(0)

    @pl.when(i == 0)
    def _init():
        accp[...] = jnp.zeros_like(accp)
        accc[...] = jnp.zeros_like(accc)

    agg = p_ref[0] + p_ref[1]
    dsum = jnp.sum(d_ref[...], axis=0)
    r = 1.0 / jnp.maximum(dsum, 1.0)
    rmat = r.reshape(RBLK // 128, 128)
    eye = (lax.broadcasted_iota(jnp.int32, (128, 128), 0)
           == lax.broadcasted_iota(jnp.int32, (128, 128), 1))
    parts = []
    for t in range(RBLK // 128):
        dt = jnp.where(eye, rmat[t:t + 1, :], 0.0)
        parts.append(jnp.dot(dt, agg[128 * t:128 * (t + 1), :],
                             preferred_element_type=jnp.float32))
    aggn = jnp.concatenate(parts, axis=0)
    h = (jnp.dot(x_ref[...], wr_ref[...], preferred_element_type=jnp.float32)
         + jnp.dot(aggn, wn_ref[...], preferred_element_type=jnp.float32)
         + bg_ref[...])
    h = jnp.maximum(h, 0.0)

    b = b_ref[0]
    gr = lax.broadcasted_iota(jnp.int32, (128, RBLK), 0)
    oh = (gr == b).astype(jnp.float32)
    accp[...] += jnp.dot(oh, h, preferred_element_type=jnp.float32)
    accc[...] += jnp.dot(oh, jnp.ones((RBLK, 128), jnp.float32),
                         preferred_element_type=jnp.float32)

    @pl.when(i == NB - 1)
    def _final():
        pooled = accp[...] / jnp.maximum(accc[...], 1.0)
        z1 = jnp.maximum(
            jnp.dot(pooled, w1_ref[...], preferred_element_type=jnp.float32)
            + b1_ref[...], 0.0)
        z = (jnp.dot(z1, w2_ref[...], preferred_element_type=jnp.float32)
             + b2_ref[...])
        lanes = lax.broadcasted_iota(jnp.int32, (128, 128), 1)
        z = jnp.where(lanes < C, z, -1e30)
        m = jnp.max(z, axis=1, keepdims=True)
        e = jnp.exp(z - m)
        sm = e / jnp.sum(e, axis=1, keepdims=True)
        o_ref[...] = sm[0:G, :]


def _tc_classify(x_pad, aggp, degp, batch3, W_root, W_nbr, b_gnn, W1, b1,
                 W2p, b2p):
    return pl.pallas_call(
        _tc_body,
        grid=(NB,),
        in_specs=[
            pl.BlockSpec((RBLK, D), lambda i: (i, 0)),
            pl.BlockSpec((NC, RBLK, D), lambda i: (0, i, 0)),
            pl.BlockSpec((NW, RBLK), lambda i: (0, i)),
            pl.BlockSpec((1, 1, RBLK), lambda i: (i, 0, 0)),
            pl.BlockSpec((D, D), lambda i: (0, 0)),
            pl.BlockSpec((D, D), lambda i: (0, 0)),
            pl.BlockSpec((D, D), lambda i: (0, 0)),
            pl.BlockSpec((D, 128), lambda i: (0, 0)),
            pl.BlockSpec((1, D), lambda i: (0, 0)),
            pl.BlockSpec((1, D), lambda i: (0, 0)),
            pl.BlockSpec((1, 128), lambda i: (0, 0)),
        ],
        out_specs=pl.BlockSpec((G, 128), lambda i: (0, 0)),
        out_shape=jax.ShapeDtypeStruct((G, 128), jnp.float32),
        scratch_shapes=[
            pltpu.VMEM((128, 128), jnp.float32),
            pltpu.VMEM((128, 128), jnp.float32),
        ],
    )(x_pad, aggp, degp, batch3, W_root, W_nbr, W1, W2p, b_gnn, b1, b2p)


def kernel(x, edge_index, batch, W_root, W_nbr, b_gnn, W1, b1, W2, b2):
    zeros_pad = jnp.zeros((ROWS_PER_TILE, D), jnp.float32)
    zeros_flat = jnp.zeros((NP,), jnp.float32)

    aggp, degp = _sc_aggregate(x, edge_index[0], edge_index[1],
                               zeros_pad, zeros_flat)

    x_pad = jnp.pad(x, ((0, NP - N), (0, 0)))
    batch3 = jnp.pad(batch, (0, NP - N),
                     constant_values=PAD_GRAPH).reshape(NB, 1, RBLK)
    W2p = jnp.pad(W2, ((0, 0), (0, 128 - C)))
    b2p = jnp.pad(b2, (0, 128 - C)).reshape(1, 128)
    out = _tc_classify(x_pad, aggp, degp, batch3, W_root, W_nbr,
                       b_gnn.reshape(1, D), W1, b1.reshape(1, D), W2p, b2p)
    return out[:, :C]
